# Initial kernel scaffold; baseline (speedup 1.0000x reference)
import jax
import jax.numpy as jnp
from jax.experimental import pallas as pl


def kernel(text, emb, w_ih, w_hh, b_ih, b_hh, w_out, b_out):
    raise NotImplementedError("write your pallas kernel here")


# speedup vs baseline: 16.9337x; 16.9337x over previous
import functools

import jax
import jax.numpy as jnp
from jax import lax
from jax.experimental import pallas as pl
from jax.experimental.pallas import tpu as pltpu
from jax.experimental.pallas import tpu_sc as plsc

_VOCAB = 100000
_E = 128
_H = 64
_NL = 10
_B = 1024
_T = 200

_NC = 2
_NS = 16
_NW = _NC * _NS
_NTOK = _B * _T
_PER_W = _NTOK // _NW
_CHUNK = 128
_NCHUNK = _PER_W // _CHUNK


def _sc_gather(emb, idx3):
    mesh = plsc.VectorSubcoreMesh(core_axis_name="c", subcore_axis_name="s")

    @functools.partial(
        pl.kernel,
        mesh=mesh,
        out_type=jax.ShapeDtypeStruct((_NTOK, _E), jnp.float32),
        scratch_types=[
            pltpu.VMEM((_NCHUNK, _CHUNK), jnp.int32),
            pltpu.VMEM((2, _CHUNK, _E), jnp.float32),
            pltpu.SemaphoreType.DMA,
            pltpu.SemaphoreType.DMA,
            pltpu.SemaphoreType.DMA,
            pltpu.SemaphoreType.DMA,
        ],
    )
    def k(emb_hbm, idx_hbm, out_hbm, idx_v, rows_v, g0, g1, s0, s1):
        wid = lax.axis_index("s") * _NC + lax.axis_index("c")
        base = wid * _PER_W
        pltpu.sync_copy(idx_hbm.at[wid], idx_v)

        gsem = (g0, g1)
        ssem = (s0, s1)

        def start_gather(j, b):
            pltpu.make_async_copy(
                emb_hbm.at[idx_v.at[j]], rows_v.at[b], gsem[b]
            ).start()

        def wait_gather(b):
            pltpu.make_async_copy(
                emb_hbm.at[idx_v.at[0]], rows_v.at[b], gsem[b]
            ).wait()

        def start_store(j, b):
            pltpu.make_async_copy(
                rows_v.at[b], out_hbm.at[pl.ds(base + j * _CHUNK, _CHUNK)], ssem[b]
            ).start()

        def wait_store(b):
            pltpu.make_async_copy(
                rows_v.at[b], out_hbm.at[pl.ds(base, _CHUNK)], ssem[b]
            ).wait()

        start_gather(0, 0)
        start_gather(1, 1)

        def body(i, carry):
            j0 = 2 * i
            wait_gather(0)
            start_store(j0, 0)
            wait_store(0)
            start_gather(j0 + 2, 0)
            wait_gather(1)
            start_store(j0 + 1, 1)
            wait_store(1)
            start_gather(j0 + 3, 1)
            return carry

        lax.fori_loop(0, _NCHUNK // 2 - 1, body, 0)
        jlast = _NCHUNK - 2
        wait_gather(0)
        start_store(jlast, 0)
        wait_store(0)
        wait_gather(1)
        start_store(jlast + 1, 1)
        wait_store(1)

    return k(emb, idx3)


def _gru_cell(gi, h, whhT, bhh):
    gh = jnp.dot(h, whhT, preferred_element_type=jnp.float32) + bhh
    ir, iz, inn = gi[:, :_H], gi[:, _H:2 * _H], gi[:, 2 * _H:]
    hr, hz, hn = gh[:, :_H], gh[:, _H:2 * _H], gh[:, 2 * _H:]
    r = jax.nn.sigmoid(ir + hr)
    z = jax.nn.sigmoid(iz + hz)
    n = jnp.tanh(inn + r * hn)
    return (1.0 - z) * n + z * h


def _l0_body(xf_ref, xb_ref, text_ref,
             wfih_ref, wfhh_ref, bfih_ref, bfhh_ref,
             wbih_ref, wbhh_ref, bbih_ref, bbhh_ref,
             outf_ref, outb_ref, hf_out, hb_out, lens_out,
             hf_s, hb_s, lens_s):
    i = pl.program_id(0)

    @pl.when(i == 0)
    def _init():
        hf_s[...] = jnp.zeros_like(hf_s)
        hb_s[...] = jnp.zeros_like(hb_s)
        tt = text_ref[...]
        lens_s[...] = jnp.sum((tt != 0).astype(jnp.int32), axis=1, keepdims=True)
        lens_out[...] = lens_s[...]

    lens = lens_s[...]

    hf = hf_s[...]
    gi_f = jnp.dot(xf_ref[0], wfih_ref[...],
                   preferred_element_type=jnp.float32) + bfih_ref[...]
    hf2 = _gru_cell(gi_f, hf, wfhh_ref[...], bfhh_ref[...])
    hf2 = jnp.where(i < lens, hf2, hf)
    hf_s[...] = hf2
    outf_ref[0] = hf2

    hb = hb_s[...]
    gi_b = jnp.dot(xb_ref[0], wbih_ref[...],
                   preferred_element_type=jnp.float32) + bbih_ref[...]
    hb2 = _gru_cell(gi_b, hb, wbhh_ref[...], bbhh_ref[...])
    hb2 = jnp.where((_T - 1 - i) < lens, hb2, hb)
    hb_s[...] = hb2
    outb_ref[0] = hb2

    @pl.when(i == _T - 1)
    def _fin():
        hf_out[...] = hf2
        hb_out[...] = hb2


def _l1_body(ff_ref, fb_ref, bf_ref, bb_ref, lens_ref, h0f_ref, h0b_ref,
             wfa_ref, wfb_ref, wfhh_ref, bfih_ref, bfhh_ref,
             wba_ref, wbb_ref, wbhh_ref, bbih_ref, bbhh_ref,
             wo0_ref, wo1_ref, wo2_ref, wo3_ref, bo_ref,
             out_ref, hf_s, hb_s):
    i = pl.program_id(0)

    @pl.when(i == 0)
    def _init():
        hf_s[...] = jnp.zeros_like(hf_s)
        hb_s[...] = jnp.zeros_like(hb_s)

    lens = lens_ref[...]

    hf = hf_s[...]
    gi_f = (jnp.dot(ff_ref[0], wfa_ref[...], preferred_element_type=jnp.float32)
            + jnp.dot(fb_ref[0], wfb_ref[...], preferred_element_type=jnp.float32)
            + bfih_ref[...])
    hf2 = _gru_cell(gi_f, hf, wfhh_ref[...], bfhh_ref[...])
    hf2 = jnp.where(i < lens, hf2, hf)
    hf_s[...] = hf2

    hb = hb_s[...]
    gi_b = (jnp.dot(bf_ref[0], wba_ref[...], preferred_element_type=jnp.float32)
            + jnp.dot(bb_ref[0], wbb_ref[...], preferred_element_type=jnp.float32)
            + bbih_ref[...])
    hb2 = _gru_cell(gi_b, hb, wbhh_ref[...], bbhh_ref[...])
    hb2 = jnp.where((_T - 1 - i) < lens, hb2, hb)
    hb_s[...] = hb2

    @pl.when(i == _T - 1)
    def _fin():
        out_ref[...] = (
            jnp.dot(h0f_ref[...], wo0_ref[...], preferred_element_type=jnp.float32)
            + jnp.dot(h0b_ref[...], wo1_ref[...], preferred_element_type=jnp.float32)
            + jnp.dot(hf2, wo2_ref[...], preferred_element_type=jnp.float32)
            + jnp.dot(hb2, wo3_ref[...], preferred_element_type=jnp.float32)
            + bo_ref[...])


def _full(shape):
    nd = len(shape)
    return pl.BlockSpec(shape, lambda i: (0,) * nd)


def kernel(text, emb, w_ih, w_hh, b_ih, b_hh, w_out, b_out):
    text = text.astype(jnp.int32)
    emb = emb.astype(jnp.float32)

    idx3 = text.T.reshape(_NW, _NCHUNK, _CHUNK)
    xT = _sc_gather(emb, idx3).reshape(_T, _B, _E)

    H3 = 3 * _H

    w0f_ihT = w_ih[0, 0].T
    w0f_hhT = w_hh[0, 0].T
    w0b_ihT = w_ih[0, 1].T
    w0b_hhT = w_hh[0, 1].T
    b0f_ih = b_ih[0, 0].reshape(1, H3)
    b0f_hh = b_hh[0, 0].reshape(1, H3)
    b0b_ih = b_ih[0, 1].reshape(1, H3)
    b0b_hh = b_hh[0, 1].reshape(1, H3)

    outf, outb, h0f, h0b, lensC = pl.pallas_call(
        _l0_body,
        grid=(_T,),
        in_specs=[
            pl.BlockSpec((1, _B, _E), lambda i: (i, 0, 0)),
            pl.BlockSpec((1, _B, _E), lambda i: (_T - 1 - i, 0, 0)),
            _full((_B, _T)),
            _full((_E, H3)), _full((_H, H3)), _full((1, H3)), _full((1, H3)),
            _full((_E, H3)), _full((_H, H3)), _full((1, H3)), _full((1, H3)),
        ],
        out_specs=[
            pl.BlockSpec((1, _B, _H), lambda i: (i, 0, 0)),
            pl.BlockSpec((1, _B, _H), lambda i: (_T - 1 - i, 0, 0)),
            _full((_B, _H)),
            _full((_B, _H)),
            _full((_B, 1)),
        ],
        out_shape=[
            jax.ShapeDtypeStruct((_T, _B, _H), jnp.float32),
            jax.ShapeDtypeStruct((_T, _B, _H), jnp.float32),
            jax.ShapeDtypeStruct((_B, _H), jnp.float32),
            jax.ShapeDtypeStruct((_B, _H), jnp.float32),
            jax.ShapeDtypeStruct((_B, 1), jnp.int32),
        ],
        scratch_shapes=[
            pltpu.VMEM((_B, _H), jnp.float32),
            pltpu.VMEM((_B, _H), jnp.float32),
            pltpu.VMEM((_B, 1), jnp.int32),
        ],
        compiler_params=pltpu.CompilerParams(
            dimension_semantics=("arbitrary",)),
    )(xT, xT, text,
      w0f_ihT, w0f_hhT, b0f_ih, b0f_hh,
      w0b_ihT, w0b_hhT, b0b_ih, b0b_hh)

    w1f_a = w_ih[1, 0][:, :_H].T
    w1f_b = w_ih[1, 0][:, _H:].T
    w1b_a = w_ih[1, 1][:, :_H].T
    w1b_b = w_ih[1, 1][:, _H:].T
    w1f_hhT = w_hh[1, 0].T
    w1b_hhT = w_hh[1, 1].T
    b1f_ih = b_ih[1, 0].reshape(1, H3)
    b1f_hh = b_hh[1, 0].reshape(1, H3)
    b1b_ih = b_ih[1, 1].reshape(1, H3)
    b1b_hh = b_hh[1, 1].reshape(1, H3)
    wo = [w_out[:, k * _H:(k + 1) * _H].T for k in range(4)]
    bo = b_out.reshape(1, _NL)

    out = pl.pallas_call(
        _l1_body,
        grid=(_T,),
        in_specs=[
            pl.BlockSpec((1, _B, _H), lambda i: (i, 0, 0)),
            pl.BlockSpec((1, _B, _H), lambda i: (i, 0, 0)),
            pl.BlockSpec((1, _B, _H), lambda i: (_T - 1 - i, 0, 0)),
            pl.BlockSpec((1, _B, _H), lambda i: (_T - 1 - i, 0, 0)),
            _full((_B, 1)),
            _full((_B, _H)), _full((_B, _H)),
            _full((_H, H3)), _full((_H, H3)), _full((_H, H3)),
            _full((1, H3)), _full((1, H3)),
            _full((_H, H3)), _full((_H, H3)), _full((_H, H3)),
            _full((1, H3)), _full((1, H3)),
            _full((_H, _NL)), _full((_H, _NL)), _full((_H, _NL)), _full((_H, _NL)),
            _full((1, _NL)),
        ],
        out_specs=pl.BlockSpec((_B, _NL), lambda i: (0, 0)),
        out_shape=jax.ShapeDtypeStruct((_B, _NL), jnp.float32),
        scratch_shapes=[
            pltpu.VMEM((_B, _H), jnp.float32),
            pltpu.VMEM((_B, _H), jnp.float32),
        ],
        compiler_params=pltpu.CompilerParams(
            dimension_semantics=("arbitrary",)),
    )(outf, outb, outf, outb, lensC, h0f, h0b,
      w1f_a, w1f_b, w1f_hhT, b1f_ih, b1f_hh,
      w1b_a, w1b_b, w1b_hhT, b1b_ih, b1b_hh,
      wo[0], wo[1], wo[2], wo[3], bo)

    return out
